# baseline (speedup 1.0000x reference)
import functools

import jax
import jax.numpy as jnp
from jax import lax
from jax.experimental import pallas as pl
from jax.experimental.pallas import tpu as pltpu
from jax.experimental.pallas import tpu_sc as plsc

N_CORES = 2
N_SUBCORES = 16
N_WORKERS = N_CORES * N_SUBCORES
LANE = 16
K = 128
CPB = 8
SK = 32
NGB = 4
NSB = 2


@functools.lru_cache(maxsize=None)
def _make_spmm(n, d, cpw):
    mesh = plsc.VectorSubcoreMesh(core_axis_name="c", subcore_axis_name="s")
    rpt = n // N_SUBCORES
    rz = 128
    assert n % N_SUBCORES == 0 and rpt % rz == 0
    nb = cpw // CPB
    assert cpw % (2 * CPB) == 0

    idx_t = [
        pltpu.VMEM((CPB, K), jnp.int32),
        pltpu.VMEM((CPB, K), jnp.int32),
        pltpu.VMEM((CPB, K), jnp.float32),
    ]

    @functools.partial(
        pl.kernel,
        out_type=jax.ShapeDtypeStruct((N_CORES, n, d), jnp.float32),
        mesh=mesh,
        scratch_types=(
            idx_t + idx_t
            + [pltpu.VMEM((SK, d), jnp.float32)] * NGB
            + [pltpu.VMEM((SK, d), jnp.float32)] * NSB
            + [pltpu.VMEM((SK,), jnp.int32)] * NSB
            + [pltpu.SemaphoreType.DMA] * NGB
            + [pltpu.SemaphoreType.DMA] * NSB
            + [pltpu.SemaphoreType.DMA] * 2
            + [pltpu.VMEM_SHARED((n, d), jnp.float32)]
        ),
    )
    def spmm(x_hbm, src_hbm, dst_hbm, val_hbm, out_hbm, *scr):
        srcbb = (scr[0], scr[3])
        dstbb = (scr[1], scr[4])
        valbb = (scr[2], scr[5])
        o = 6
        gbuf = scr[o:o + NGB]; o += NGB
        sbuf = scr[o:o + NSB]; o += NSB
        dstq = scr[o:o + NSB]; o += NSB
        gsem = scr[o:o + NGB]; o += NGB
        ssem = scr[o:o + NSB]; o += NSB
        isem = scr[o:o + 2]; o += 2

        cid = lax.axis_index("c")
        sid = lax.axis_index("s")
        wid = cid * N_SUBCORES + sid
        crow0 = wid * cpw

        acc = scr[o]

        def zbody(r, carry):
            for j in range(d // LANE):
                gbuf[0][r, pl.ds(j * LANE, LANE)] = jnp.zeros(
                    (LANE,), jnp.float32)
            return carry
        lax.fori_loop(0, SK, zbody, 0)
        zbase = sid * rpt
        for i in range(rpt // SK):
            pltpu.sync_copy(gbuf[0].at[pl.ds(0, SK)],
                            acc.at[pl.ds(zbase + i * SK, SK)])

        def load_block(b, h):
            r0 = crow0 + b * CPB
            pltpu.async_copy(src_hbm.at[pl.ds(r0, CPB)], srcbb[h], isem[h])
            pltpu.async_copy(dst_hbm.at[pl.ds(r0, CPB)], dstbb[h], isem[h])
            pltpu.async_copy(val_hbm.at[pl.ds(r0, CPB)], valbb[h], isem[h])

        def wait_block(h):
            pltpu.make_async_copy(
                src_hbm.at[pl.ds(0, CPB)], srcbb[h], isem[h]).wait()
            pltpu.make_async_copy(
                dst_hbm.at[pl.ds(0, CPB)], dstbb[h], isem[h]).wait()
            pltpu.make_async_copy(
                val_hbm.at[pl.ds(0, CPB)], valbb[h], isem[h]).wait()

        def scale_sub(h, ci, q, gb, sb):
            for g in range(SK // LANE):
                vv = valbb[h][ci, pl.ds(q * SK + g * LANE, LANE)]
                for i in range(LANE):
                    r = g * LANE + i
                    v = vv[i]
                    tmp = [gb[r, pl.ds(j * LANE, LANE)]
                           for j in range(d // LANE)]
                    for j in range(d // LANE):
                        sb[r, pl.ds(j * LANE, LANE)] = tmp[j] * v

        def issue_gather(h, ci, q, k):
            return pltpu.async_copy(
                x_hbm.at[srcbb[h].at[ci, pl.ds(q * SK, SK)]],
                gbuf[k], gsem[k])

        def wait_gather(h, k):
            pltpu.make_async_copy(
                x_hbm.at[srcbb[h].at[0, pl.ds(0, SK)]],
                gbuf[k], gsem[k]).wait()

        def wait_scatter(s):
            pltpu.make_async_copy(
                sbuf[s], acc.at[dstq[s]], ssem[s]).wait()

        def process_block(h):
            for k in range(NGB):
                issue_gather(h, 0, k, k)

            def body(ci, carry):
                for k in range(NGB):
                    s = k % NSB
                    wait_gather(h, k)
                    if k >= NSB:
                        wait_scatter(s)
                    else:
                        @pl.when(ci > 0)
                        def _():
                            wait_scatter(s)
                    scale_sub(h, ci, k, gbuf[k], sbuf[s])
                    for q16 in range(SK // LANE):
                        sl = pl.ds(q16 * LANE, LANE)
                        dstq[s][sl] = dstbb[h][ci, pl.ds(
                            k * SK + q16 * LANE, LANE)]
                    pltpu.async_copy(
                        sbuf[s], acc.at[dstq[s]], ssem[s], add=True)

                    @pl.when(ci + 1 < CPB)
                    def _():
                        issue_gather(h, ci + 1, k, k)
                return carry

            lax.fori_loop(0, CPB, body, 0)
            for s in range(NSB):
                wait_scatter(s)

        load_block(0, 0)

        def pairbody(t, carry):
            b0 = 2 * t
            load_block(b0 + 1, 1)
            wait_block(0)
            process_block(0)

            @pl.when(b0 + 2 < nb)
            def _():
                load_block(b0 + 2, 0)

            wait_block(1)
            process_block(1)
            return carry

        lax.fori_loop(0, nb // 2, pairbody, 0)
        plsc.subcore_barrier()

        for i in range(rpt // rz):
            r = zbase + i * rz
            pltpu.sync_copy(acc.at[pl.ds(r, rz)],
                            out_hbm.at[cid, pl.ds(r, rz)])

    return spmm


def _combine_matmul(p, w, bias, n):
    d = p.shape[2]
    d_out = w.shape[1]
    bm = 400
    assert n % bm == 0

    def body(p_ref, w_ref, b_ref, o_ref):
        s = p_ref[0] + p_ref[1]
        o_ref[...] = jnp.dot(
            s, w_ref[...], preferred_element_type=jnp.float32) + b_ref[...]

    return pl.pallas_call(
        body,
        grid=(n // bm,),
        in_specs=[
            pl.BlockSpec((2, bm, d), lambda i: (0, i, 0)),
            pl.BlockSpec((d, d_out), lambda i: (0, 0)),
            pl.BlockSpec((1, d_out), lambda i: (0, 0)),
        ],
        out_specs=pl.BlockSpec((bm, d_out), lambda i: (i, 0)),
        out_shape=jax.ShapeDtypeStruct((n, d_out), jnp.float32),
    )(p, w, bias.reshape(1, d_out))


def kernel(x, edge_index, edge_vals, W, bias):
    n, _ = x.shape
    e = edge_vals.shape[0]
    src = edge_index[0].astype(jnp.int32)
    dst = edge_index[1].astype(jnp.int32)
    vals = edge_vals.astype(jnp.float32)

    quantum = N_WORKERS * K * 2 * CPB
    e_pad = -(-e // quantum) * quantum
    cpw = e_pad // (N_WORKERS * K)
    if e_pad > e:
        pad = e_pad - e
        src = jnp.concatenate([src, jnp.zeros((pad,), jnp.int32)])
        dst = jnp.concatenate([dst, jnp.zeros((pad,), jnp.int32)])
        vals = jnp.concatenate([vals, jnp.zeros((pad,), jnp.float32)])

    src2 = src.reshape(-1, K)
    dst2 = dst.reshape(-1, K)
    val2 = vals.reshape(-1, K)

    n_pad = -(-n // (128 * N_SUBCORES)) * (128 * N_SUBCORES)
    partials = _make_spmm(n_pad, x.shape[1], cpw)(x, src2, dst2, val2)
    return _combine_matmul(partials, W, bias, n)


# speedup vs baseline: 1.6900x; 1.6900x over previous
import functools

import jax
import jax.numpy as jnp
from jax import lax
from jax.experimental import pallas as pl
from jax.experimental.pallas import tpu as pltpu
from jax.experimental.pallas import tpu_sc as plsc

N_CORES = 2
N_SUBCORES = 16
N_WORKERS = N_CORES * N_SUBCORES
LANE = 16
SK = 16
UPB = 16
EPB = SK * UPB


def _to_bf16_bits(v):
    u = lax.bitcast_convert_type(v, jnp.int32)
    rnd = jnp.int32(0x7FFF) + ((u >> 16) & 1)
    return (u + rnd) >> 16


@functools.lru_cache(maxsize=None)
def _make_spmm(n, d, epw):
    mesh = plsc.VectorSubcoreMesh(core_axis_name="c", subcore_axis_name="s")
    rpt = n // N_SUBCORES
    rz = 128
    assert n % N_SUBCORES == 0 and rpt % rz == 0 and rpt % SK == 0
    nb = epw // EPB
    assert epw % (2 * EPB) == 0
    dw = d // 2

    idx_t = [
        pltpu.VMEM((EPB,), jnp.int32),
        pltpu.VMEM((EPB,), jnp.int32),
        pltpu.VMEM((EPB,), jnp.float32),
    ]

    @functools.partial(
        pl.kernel,
        out_type=jax.ShapeDtypeStruct((N_CORES, n, d), jnp.float32),
        mesh=mesh,
        scratch_types=(
            idx_t + idx_t
            + [pltpu.VMEM((SK, dw), jnp.int32)]
            + [pltpu.VMEM((SK, d), jnp.float32)] * 2
            + [pltpu.VMEM((SK,), jnp.int32)] * 2
            + [pltpu.SemaphoreType.DMA]
            + [pltpu.SemaphoreType.DMA] * 2
            + [pltpu.SemaphoreType.DMA] * 2
            + [pltpu.VMEM_SHARED((n, dw), jnp.int32)]
            + [pltpu.VMEM_SHARED((n, d), jnp.float32)]
        ),
    )
    def spmm(x_hbm, src_hbm, dst_hbm, val_hbm, out_hbm, *scr):
        srcb = (scr[0], scr[3])
        dstb = (scr[1], scr[4])
        valb = (scr[2], scr[5])
        gbuf = scr[6]
        sbuf = scr[7:9]
        dstq = scr[9:11]
        gsem = scr[11]
        ssem = scr[12:14]
        isem = scr[14:16]
        xs = scr[16]
        acc = scr[17]

        cid = lax.axis_index("c")
        sid = lax.axis_index("s")
        wid = cid * N_SUBCORES + sid
        ebase = wid * epw
        zbase = sid * rpt

        def zb(r, carry):
            for j in range(d // LANE):
                sbuf[0][r, pl.ds(j * LANE, LANE)] = jnp.zeros(
                    (LANE,), jnp.float32)
            return carry
        lax.fori_loop(0, SK, zb, 0)
        for i in range(rpt // SK):
            pltpu.sync_copy(sbuf[0].at[pl.ds(0, SK)],
                            acc.at[pl.ds(zbase + i * SK, SK)])

        def stage(i, carry):
            pltpu.sync_copy(x_hbm.at[pl.ds(zbase + i * SK, SK)], sbuf[0])
            for r in range(SK):
                for g in range(d // (2 * LANE)):
                    a = sbuf[0][r, pl.ds(g * 2 * LANE, LANE)]
                    b = sbuf[0][r, pl.ds(g * 2 * LANE + LANE, LANE)]
                    w = (_to_bf16_bits(a) & jnp.int32(0xFFFF)) | (
                        _to_bf16_bits(b) << 16)
                    gbuf[r, pl.ds(g * LANE, LANE)] = w
            pltpu.sync_copy(gbuf, xs.at[pl.ds(zbase + i * SK, SK)])
            return carry
        lax.fori_loop(0, rpt // SK, stage, 0)
        plsc.subcore_barrier()

        def issue_gather(h, u):
            return pltpu.async_copy(
                xs.at[srcb[h].at[pl.ds(u * SK, SK)]], gbuf, gsem)

        def wait_gather(h):
            pltpu.make_async_copy(
                xs.at[srcb[h].at[pl.ds(0, SK)]], gbuf, gsem).wait()

        def wait_scatter(k):
            pltpu.make_async_copy(
                sbuf[k], acc.at[dstq[k]], ssem[k]).wait()

        def unpack_scale(h, u, k):
            vv = valb[h][pl.ds(u * SK, SK)]
            for r in range(SK):
                v = vv[r]
                for g in range(d // (2 * LANE)):
                    w = gbuf[r, pl.ds(g * LANE, LANE)]
                    a = lax.bitcast_convert_type(w << 16, jnp.float32)
                    b = lax.bitcast_convert_type(
                        w & jnp.int32(-65536), jnp.float32)
                    sbuf[k][r, pl.ds(g * 2 * LANE, LANE)] = a * v
                    sbuf[k][r, pl.ds(g * 2 * LANE + LANE, LANE)] = b * v

        def load_block(b, h):
            off = ebase + b * EPB
            pltpu.async_copy(src_hbm.at[pl.ds(off, EPB)], srcb[h], isem[h])
            pltpu.async_copy(dst_hbm.at[pl.ds(off, EPB)], dstb[h], isem[h])
            pltpu.async_copy(val_hbm.at[pl.ds(off, EPB)], valb[h], isem[h])

        def wait_block(h):
            pltpu.make_async_copy(
                src_hbm.at[pl.ds(0, EPB)], srcb[h], isem[h]).wait()
            pltpu.make_async_copy(
                dst_hbm.at[pl.ds(0, EPB)], dstb[h], isem[h]).wait()
            pltpu.make_async_copy(
                val_hbm.at[pl.ds(0, EPB)], valb[h], isem[h]).wait()

        def process_block(h):
            issue_gather(h, 0)

            def body(uu, carry):
                for k in range(2):
                    u = 2 * uu + k
                    wait_gather(h)

                    @pl.when(uu > 0)
                    def _():
                        wait_scatter(k)

                    unpack_scale(h, u, k)

                    @pl.when(u + 1 < UPB)
                    def _():
                        issue_gather(h, u + 1)

                    dstq[k][pl.ds(0, SK)] = dstb[h][pl.ds(u * SK, SK)]
                    pltpu.async_copy(
                        sbuf[k], acc.at[dstq[k]], ssem[k], add=True)
                return carry

            lax.fori_loop(0, UPB // 2, body, 0)
            wait_scatter(0)
            wait_scatter(1)

        load_block(0, 0)

        def pairbody(t, carry):
            b0 = 2 * t
            load_block(b0 + 1, 1)
            wait_block(0)
            process_block(0)

            @pl.when(b0 + 2 < nb)
            def _():
                load_block(b0 + 2, 0)

            wait_block(1)
            process_block(1)
            return carry

        lax.fori_loop(0, nb // 2, pairbody, 0)
        plsc.subcore_barrier()

        for i in range(rpt // rz):
            r = zbase + i * rz
            pltpu.sync_copy(acc.at[pl.ds(r, rz)],
                            out_hbm.at[cid, pl.ds(r, rz)])

    return spmm


def _combine_matmul(p, w, bias, n):
    d = p.shape[2]
    d_out = w.shape[1]
    bm = 400
    assert n % bm == 0

    def body(p_ref, w_ref, b_ref, o_ref):
        s = p_ref[0] + p_ref[1]
        o_ref[...] = jnp.dot(
            s, w_ref[...], preferred_element_type=jnp.float32) + b_ref[...]

    return pl.pallas_call(
        body,
        grid=(n // bm,),
        in_specs=[
            pl.BlockSpec((2, bm, d), lambda i: (0, i, 0)),
            pl.BlockSpec((d, d_out), lambda i: (0, 0)),
            pl.BlockSpec((1, d_out), lambda i: (0, 0)),
        ],
        out_specs=pl.BlockSpec((bm, d_out), lambda i: (i, 0)),
        out_shape=jax.ShapeDtypeStruct((n, d_out), jnp.float32),
    )(p, w, bias.reshape(1, d_out))


def kernel(x, edge_index, edge_vals, W, bias):
    n, d = x.shape
    e = edge_vals.shape[0]
    src = edge_index[0].astype(jnp.int32)
    dst = edge_index[1].astype(jnp.int32)
    vals = edge_vals.astype(jnp.float32)

    quantum = N_WORKERS * 2 * EPB
    e_pad = -(-e // quantum) * quantum
    epw = e_pad // N_WORKERS
    if e_pad > e:
        pad = e_pad - e
        src = jnp.concatenate([src, jnp.zeros((pad,), jnp.int32)])
        dst = jnp.concatenate([dst, jnp.zeros((pad,), jnp.int32)])
        vals = jnp.concatenate([vals, jnp.zeros((pad,), jnp.float32)])

    n_pad = -(-n // (128 * N_SUBCORES)) * (128 * N_SUBCORES)
    if n_pad > n:
        x = jnp.pad(x, ((0, n_pad - n), (0, 0)))

    partials = _make_spmm(n_pad, d, epw)(x, src, dst, vals)
    return _combine_matmul(partials, W, bias, n)
